# baseline (speedup 1.0000x reference)
import jax
import jax.numpy as jnp
from jax import lax
from jax.experimental import pallas as pl
from jax.experimental.pallas import tpu as pltpu
from jax.experimental.pallas import tpu_sc as plsc

_INFO = plsc.get_sparse_core_info()
_NC, _NS, _L = _INFO.num_cores, _INFO.num_subcores, _INFO.num_lanes
_NW = _NC * _NS

_D = 32
_B = 16384 * 50
_BPW = _B // _NW
_CHUNK = 512
_NCHUNK = _BPW // _CHUNK
_GATHER_W = 128
_NGATHER = _CHUNK // _GATHER_W
_GROUPS = _CHUNK // _L


def _rsqrt16(s):
    i = plsc.bitcast(s, jnp.int32)
    i = jnp.int32(0x5F3759DF) - (i >> 1)
    y = plsc.bitcast(i, jnp.float32)
    for _ in range(3):
        y = y * (jnp.float32(1.5) - jnp.float32(0.5) * s * y * y)
    return y


def _emb_body(idx_hbm, table_hbm, out_hbm,
              idx0, idx1, rows0, rows1, tr0, tr1,
              g0, g1, o0, o1):
    wid = lax.axis_index("s") * _NC + lax.axis_index("c")
    base = wid * _BPW
    cbase = wid * _NCHUNK
    iota = lax.iota(jnp.int32, _L)

    def fire(ci, idx_v, rows_v, gsem):
        pltpu.sync_copy(idx_hbm.at[cbase + ci], idx_v)
        for j in range(_NGATHER):
            pltpu.async_copy(
                table_hbm.at[idx_v.at[j]],
                rows_v.at[pl.ds(j * _GATHER_W, _GATHER_W)],
                gsem,
            )

    def wait_gather(rows_v, gsem):
        pltpu.make_async_copy(
            table_hbm.at[pl.ds(0, _CHUNK)], rows_v, gsem).wait()

    def compute(rows_v, tr_v):
        def group_body(g, c2):
            rowv = g * _L + iota
            cols = []
            s = None
            for d in range(_D):
                colv = jnp.full((_L,), d, jnp.int32)
                c = plsc.load_gather(rows_v, [rowv, colv])
                cols.append(c)
                s = c * c if s is None else s + c * c
            inv = _rsqrt16(s)
            for d in range(_D):
                tr_v[d, pl.ds(g * _L, _L)] = cols[d] * inv
            return c2

        lax.fori_loop(0, _GROUPS, group_body, 0)

    def fire_out(ci, tr_v, osem):
        pltpu.async_copy(
            tr_v, out_hbm.at[:, pl.ds(base + ci * _CHUNK, _CHUNK)], osem)

    def wait_out(tr_v, osem):
        pltpu.make_async_copy(
            tr_v, out_hbm.at[:, pl.ds(0, _CHUNK)], osem).wait()

    bufs = ((idx0, rows0, tr0, g0, o0), (idx1, rows1, tr1, g1, o1))

    fire(0, idx0, rows0, g0)

    def pair_body(i, carry):
        for b in range(2):
            idx_v, rows_v, tr_v, gsem, osem = bufs[b]
            nidx_v, nrows_v, _, ngsem, _ = bufs[1 - b]
            ci = 2 * i + b

            @pl.when(ci + 1 < _NCHUNK)
            def _():
                fire(ci + 1, nidx_v, nrows_v, ngsem)

            wait_gather(rows_v, gsem)

            @pl.when(ci >= 2)
            def _():
                wait_out(tr_v, osem)

            compute(rows_v, tr_v)
            fire_out(ci, tr_v, osem)
        return carry

    lax.fori_loop(0, _NCHUNK // 2, pair_body, 0)
    wait_out(tr0, o0)
    wait_out(tr1, o1)


@jax.jit
def _emb(idx3d, table):
    mesh = plsc.VectorSubcoreMesh(core_axis_name="c", subcore_axis_name="s")
    f = pl.kernel(
        _emb_body,
        out_type=jax.ShapeDtypeStruct((_D, _B), jnp.float32),
        mesh=mesh,
        scratch_types=[
            pltpu.VMEM((_NGATHER, _GATHER_W), jnp.int32),
            pltpu.VMEM((_NGATHER, _GATHER_W), jnp.int32),
            pltpu.VMEM((_CHUNK, _D), jnp.float32),
            pltpu.VMEM((_CHUNK, _D), jnp.float32),
            pltpu.VMEM((_D, _CHUNK), jnp.float32),
            pltpu.VMEM((_D, _CHUNK), jnp.float32),
            pltpu.SemaphoreType.DMA,
            pltpu.SemaphoreType.DMA,
            pltpu.SemaphoreType.DMA,
            pltpu.SemaphoreType.DMA,
        ],
        compiler_params=pltpu.CompilerParams(
            needs_layout_passes=False, use_tc_tiling_on_sc=False),
    )
    return f(idx3d, table)


def kernel(inputs, table):
    idx = inputs.reshape(-1).astype(jnp.int32).reshape(
        _B // _CHUNK, _NGATHER, _GATHER_W)
    out_t = _emb(idx, table)
    return out_t.T.reshape(inputs.shape + (_D,))


# speedup vs baseline: 1.8476x; 1.8476x over previous
import jax
import jax.numpy as jnp
from jax import lax
from jax.experimental import pallas as pl
from jax.experimental.pallas import tpu as pltpu
from jax.experimental.pallas import tpu_sc as plsc

_INFO = plsc.get_sparse_core_info()
_NC, _NS, _L = _INFO.num_cores, _INFO.num_subcores, _INFO.num_lanes
_NW = _NC * _NS

_D = 32
_B = 16384 * 50
_BPW = _B // _NW
_CHUNK = 512
_NCHUNK = _BPW // _CHUNK
_GATHER_W = 128
_NGATHER = _CHUNK // _GATHER_W
_GROUPS = _CHUNK // _L


def _rsqrt16(s):
    i = plsc.bitcast(s, jnp.int32)
    i = jnp.int32(0x5F3759DF) - (i >> 1)
    y = plsc.bitcast(i, jnp.float32)
    for _ in range(3):
        y = y * (jnp.float32(1.5) - jnp.float32(0.5) * s * y * y)
    return y


def _emb_body(idx_hbm, table_hbm, out_hbm,
              idx0, idx1, rows0, rows1, st0, st1,
              g0, g1, o0, o1):
    wid = lax.axis_index("s") * _NC + lax.axis_index("c")
    base = wid * _BPW
    cbase = wid * _NCHUNK
    iota = lax.iota(jnp.int32, _L)

    def fire(ci, idx_v, rows_v, gsem):
        pltpu.sync_copy(idx_hbm.at[cbase + ci], idx_v)
        for j in range(_NGATHER):
            pltpu.async_copy(
                table_hbm.at[idx_v.at[j]],
                rows_v.at[pl.ds(j * _GATHER_W, _GATHER_W)],
                gsem,
            )

    def wait_gather(rows_v, gsem):
        pltpu.make_async_copy(
            table_hbm.at[pl.ds(0, _CHUNK)], rows_v, gsem).wait()

    def compute(rows_v, st_v):
        def group_body(g, c2):
            rowv = g * _L + iota
            cols = []
            s = None
            for d in range(_D):
                colv = jnp.full((_L,), d, jnp.int32)
                c = plsc.load_gather(rows_v, [rowv, colv])
                cols.append(c)
                s = c * c if s is None else s + c * c
            inv = _rsqrt16(s)
            for d in range(_D):
                colv = jnp.full((_L,), d, jnp.int32)
                plsc.store_scatter(st_v, [rowv, colv], cols[d] * inv)
            return c2

        lax.fori_loop(0, _GROUPS, group_body, 0)

    def fire_out(ci, st_v, osem):
        pltpu.async_copy(
            st_v, out_hbm.at[pl.ds(base + ci * _CHUNK, _CHUNK)], osem)

    def wait_out(st_v, osem):
        pltpu.make_async_copy(
            st_v, out_hbm.at[pl.ds(0, _CHUNK)], osem).wait()

    bufs = ((idx0, rows0, st0, g0, o0), (idx1, rows1, st1, g1, o1))

    fire(0, idx0, rows0, g0)

    def pair_body(i, carry):
        for b in range(2):
            idx_v, rows_v, st_v, gsem, osem = bufs[b]
            nidx_v, nrows_v, _, ngsem, _ = bufs[1 - b]
            ci = 2 * i + b

            @pl.when(ci + 1 < _NCHUNK)
            def _():
                fire(ci + 1, nidx_v, nrows_v, ngsem)

            wait_gather(rows_v, gsem)

            @pl.when(ci >= 2)
            def _():
                wait_out(st_v, osem)

            compute(rows_v, st_v)
            fire_out(ci, st_v, osem)
        return carry

    lax.fori_loop(0, _NCHUNK // 2, pair_body, 0)
    wait_out(st0, o0)
    wait_out(st1, o1)


@jax.jit
def _emb(idx3d, table):
    mesh = plsc.VectorSubcoreMesh(core_axis_name="c", subcore_axis_name="s")
    f = pl.kernel(
        _emb_body,
        out_type=jax.ShapeDtypeStruct((_B, _D), jnp.float32),
        mesh=mesh,
        scratch_types=[
            pltpu.VMEM((_NGATHER, _GATHER_W), jnp.int32),
            pltpu.VMEM((_NGATHER, _GATHER_W), jnp.int32),
            pltpu.VMEM((_CHUNK, _D), jnp.float32),
            pltpu.VMEM((_CHUNK, _D), jnp.float32),
            pltpu.VMEM((_CHUNK, _D), jnp.float32),
            pltpu.VMEM((_CHUNK, _D), jnp.float32),
            pltpu.SemaphoreType.DMA,
            pltpu.SemaphoreType.DMA,
            pltpu.SemaphoreType.DMA,
            pltpu.SemaphoreType.DMA,
        ],
        compiler_params=pltpu.CompilerParams(
            needs_layout_passes=False, use_tc_tiling_on_sc=False),
    )
    return f(idx3d, table)


def kernel(inputs, table):
    idx = inputs.reshape(-1).astype(jnp.int32).reshape(
        _B // _CHUNK, _NGATHER, _GATHER_W)
    out = _emb(idx, table)
    return out.reshape(inputs.shape + (_D,))


# speedup vs baseline: 2.4101x; 1.3045x over previous
import jax
import jax.numpy as jnp
from jax import lax
from jax.experimental import pallas as pl
from jax.experimental.pallas import tpu as pltpu
from jax.experimental.pallas import tpu_sc as plsc

_INFO = plsc.get_sparse_core_info()
_NC, _NS, _L = _INFO.num_cores, _INFO.num_subcores, _INFO.num_lanes
_NW = _NC * _NS

_D = 32
_B = 16384 * 50
_BPW = _B // _NW
_CHUNK = 512
_NCHUNK = _BPW // _CHUNK
_GATHER_W = 128
_NGATHER = _CHUNK // _GATHER_W
_GROUPS = _CHUNK // _L


def _rsqrt16(s):
    i = plsc.bitcast(s, jnp.int32)
    i = jnp.int32(0x5F3759DF) - (i >> 1)
    y = plsc.bitcast(i, jnp.float32)
    for _ in range(3):
        y = y * (jnp.float32(1.5) - jnp.float32(0.5) * s * y * y)
    return y


def _emb_body(idx_hbm, table_hbm, out_hbm,
              idx0, idx1, rows0, rows1, st0, st1,
              g0, g1, o0, o1):
    wid = lax.axis_index("s") * _NC + lax.axis_index("c")
    base = wid * _BPW
    cbase = wid * _NCHUNK
    iota = lax.iota(jnp.int32, _L)

    def fire(ci, idx_v, rows_v, gsem):
        pltpu.sync_copy(idx_hbm.at[cbase + ci], idx_v)
        for j in range(_NGATHER):
            pltpu.async_copy(
                table_hbm.at[idx_v.at[j]],
                rows_v.at[pl.ds(j * _GATHER_W, _GATHER_W)],
                gsem,
            )

    def wait_gather(rows_v, gsem):
        pltpu.make_async_copy(
            table_hbm.at[pl.ds(0, _CHUNK)], rows_v, gsem).wait()

    def compute(rows_v, st_v):
        def group_body(g, c2):
            rowv = g * _L + iota
            diags = []
            colvs = []
            s = None
            for d in range(_D):
                colv = (iota + d) & (_D - 1)
                c = plsc.load_gather(rows_v, [rowv, colv])
                diags.append(c)
                colvs.append(colv)
                s = c * c if s is None else s + c * c
            inv = _rsqrt16(s)
            for d in range(_D):
                plsc.store_scatter(st_v, [rowv, colvs[d]], diags[d] * inv)
            return c2

        lax.fori_loop(0, _GROUPS, group_body, 0)

    def fire_out(ci, st_v, osem):
        pltpu.async_copy(
            st_v, out_hbm.at[pl.ds(base + ci * _CHUNK, _CHUNK)], osem)

    def wait_out(st_v, osem):
        pltpu.make_async_copy(
            st_v, out_hbm.at[pl.ds(0, _CHUNK)], osem).wait()

    bufs = ((idx0, rows0, st0, g0, o0), (idx1, rows1, st1, g1, o1))

    fire(0, idx0, rows0, g0)

    def pair_body(i, carry):
        for b in range(2):
            idx_v, rows_v, st_v, gsem, osem = bufs[b]
            nidx_v, nrows_v, _, ngsem, _ = bufs[1 - b]
            ci = 2 * i + b

            @pl.when(ci + 1 < _NCHUNK)
            def _():
                fire(ci + 1, nidx_v, nrows_v, ngsem)

            wait_gather(rows_v, gsem)

            @pl.when(ci >= 2)
            def _():
                wait_out(st_v, osem)

            compute(rows_v, st_v)
            fire_out(ci, st_v, osem)
        return carry

    lax.fori_loop(0, _NCHUNK // 2, pair_body, 0)
    wait_out(st0, o0)
    wait_out(st1, o1)


@jax.jit
def _emb(idx3d, table):
    mesh = plsc.VectorSubcoreMesh(core_axis_name="c", subcore_axis_name="s")
    f = pl.kernel(
        _emb_body,
        out_type=jax.ShapeDtypeStruct((_B, _D), jnp.float32),
        mesh=mesh,
        scratch_types=[
            pltpu.VMEM((_NGATHER, _GATHER_W), jnp.int32),
            pltpu.VMEM((_NGATHER, _GATHER_W), jnp.int32),
            pltpu.VMEM((_CHUNK, _D), jnp.float32),
            pltpu.VMEM((_CHUNK, _D), jnp.float32),
            pltpu.VMEM((_CHUNK, _D), jnp.float32),
            pltpu.VMEM((_CHUNK, _D), jnp.float32),
            pltpu.SemaphoreType.DMA,
            pltpu.SemaphoreType.DMA,
            pltpu.SemaphoreType.DMA,
            pltpu.SemaphoreType.DMA,
        ],
        compiler_params=pltpu.CompilerParams(
            needs_layout_passes=False, use_tc_tiling_on_sc=False),
    )
    return f(idx3d, table)


def kernel(inputs, table):
    idx = inputs.reshape(-1).astype(jnp.int32).reshape(
        _B // _CHUNK, _NGATHER, _GATHER_W)
    out = _emb(idx, table)
    return out.reshape(inputs.shape + (_D,))


# speedup vs baseline: 5.4877x; 2.2770x over previous
import jax
import jax.numpy as jnp
from jax import lax
from jax.experimental import pallas as pl
from jax.experimental.pallas import tpu as pltpu
from jax.experimental.pallas import tpu_sc as plsc

_INFO = plsc.get_sparse_core_info()
_NC, _NS, _L = _INFO.num_cores, _INFO.num_subcores, _INFO.num_lanes
_NW = _NC * _NS

_D = 32
_B0 = 16384
_B1 = 50
_BW = 128
_NB = _B0 // _BW
_NU = _B1 * _NB
_UPM = 4
_NM = _NU // (_NW * _UPM)
_GROUPS = _BW // _L


def _rsqrt16(s):
    i = plsc.bitcast(s, jnp.int32)
    i = jnp.int32(0x5F3759DF) - (i >> 1)
    y = plsc.bitcast(i, jnp.float32)
    for _ in range(3):
        y = y * (jnp.float32(1.5) - jnp.float32(0.5) * s * y * y)
    return y


def _emb_body(idxT_hbm, table_hbm, out_hbm,
              idx0, idx1, rows0, rows1, st0, st1,
              g0, g1, o0, o1):
    wid = lax.axis_index("s") * _NC + lax.axis_index("c")
    ubase = wid * _UPM * _NM
    iota = lax.iota(jnp.int32, _L)

    def fire(m, idx_v, rows_v, gsem):
        for j in range(_UPM):
            u = ubase + m * _UPM + j
            i1 = u // _NB
            b = u % _NB
            pltpu.sync_copy(idxT_hbm.at[i1, pl.ds(b * _BW, _BW)],
                            idx_v.at[j])
            pltpu.async_copy(
                table_hbm.at[idx_v.at[j]],
                rows_v.at[pl.ds(j * _BW, _BW)],
                gsem,
            )

    def wait_gather(rows_v, gsem):
        pltpu.make_async_copy(
            table_hbm.at[pl.ds(0, _UPM * _BW)], rows_v, gsem).wait()

    def compute(rows_v, st_v):
        def group_body(g, c2):
            rowv = g * _L + iota
            j = g // _GROUPS
            rloc = rowv - j * _BW
            diags = []
            colvs = []
            s = None
            for d in range(_D):
                colv = (iota + d) & (_D - 1)
                c = plsc.load_gather(rows_v, [rowv, colv])
                diags.append(c)
                colvs.append(colv)
                s = c * c if s is None else s + c * c
            inv = _rsqrt16(s)
            for d in range(_D):
                plsc.store_scatter(
                    st_v, [jnp.full((_L,), j, jnp.int32), colvs[d], rloc],
                    diags[d] * inv)
            return c2

        lax.fori_loop(0, _UPM * _GROUPS, group_body, 0)

    def fire_out(m, st_v, osem):
        for j in range(_UPM):
            u = ubase + m * _UPM + j
            i1 = u // _NB
            b = u % _NB
            for a in range(_D // 8):
                pltpu.async_copy(
                    st_v.at[j, pl.ds(a * 8, 8)],
                    out_hbm.at[i1, a, b],
                    osem,
                )

    def wait_out(st_v, osem):
        for j in range(_UPM):
            for a in range(_D // 8):
                pltpu.make_async_copy(
                    st_v.at[j, pl.ds(a * 8, 8)],
                    out_hbm.at[0, a, 0],
                    osem,
                ).wait()

    bufs = ((idx0, rows0, st0, g0, o0), (idx1, rows1, st1, g1, o1))

    fire(0, idx0, rows0, g0)

    def pair_body(i, carry):
        for b in range(2):
            idx_v, rows_v, st_v, gsem, osem = bufs[b]
            nidx_v, nrows_v, _, ngsem, _ = bufs[1 - b]
            m = 2 * i + b

            @pl.when(m + 1 < _NM)
            def _():
                fire(m + 1, nidx_v, nrows_v, ngsem)

            wait_gather(rows_v, gsem)

            @pl.when(m >= 2)
            def _():
                wait_out(st_v, osem)

            compute(rows_v, st_v)
            fire_out(m, st_v, osem)
        return carry

    lax.fori_loop(0, _NM // 2, pair_body, 0)
    wait_out(st0, o0)
    wait_out(st1, o1)


@jax.jit
def _emb(idxT, table):
    mesh = plsc.VectorSubcoreMesh(core_axis_name="c", subcore_axis_name="s")
    f = pl.kernel(
        _emb_body,
        out_type=jax.ShapeDtypeStruct((_B1, _D // 8, _NB, 8, _BW),
                                      jnp.float32),
        mesh=mesh,
        scratch_types=[
            pltpu.VMEM((_UPM, _BW), jnp.int32),
            pltpu.VMEM((_UPM, _BW), jnp.int32),
            pltpu.VMEM((_UPM * _BW, _D), jnp.float32),
            pltpu.VMEM((_UPM * _BW, _D), jnp.float32),
            pltpu.VMEM((_UPM, _D, _BW), jnp.float32),
            pltpu.VMEM((_UPM, _D, _BW), jnp.float32),
            pltpu.SemaphoreType.DMA,
            pltpu.SemaphoreType.DMA,
            pltpu.SemaphoreType.DMA,
            pltpu.SemaphoreType.DMA,
        ],
        compiler_params=pltpu.CompilerParams(
            needs_layout_passes=False, use_tc_tiling_on_sc=False),
    )
    return f(idxT, table)


def kernel(inputs, table):
    idxT = inputs.astype(jnp.int32).T
    y5 = _emb(idxT, table)
    return y5.transpose(2, 4, 0, 1, 3).reshape(_B0, _B1, _D)


# speedup vs baseline: 6.4075x; 1.1676x over previous
import jax
import jax.numpy as jnp
from jax import lax
from jax.experimental import pallas as pl
from jax.experimental.pallas import tpu as pltpu
from jax.experimental.pallas import tpu_sc as plsc

_INFO = plsc.get_sparse_core_info()
_NC, _NS, _L = _INFO.num_cores, _INFO.num_subcores, _INFO.num_lanes
_NW = _NC * _NS

_D = 32
_B0 = 16384
_B1 = 50
_BW = 128
_NB = _B0 // _BW
_NU = _B1 * _NB
_UPM = 4
_NM = _NU // (_NW * _UPM)
_GROUPS = _BW // _L


def _rsqrt16(s):
    i = plsc.bitcast(s, jnp.int32)
    i = jnp.int32(0x5F3759DF) - (i >> 1)
    y = plsc.bitcast(i, jnp.float32)
    for _ in range(3):
        y = y * (jnp.float32(1.5) - jnp.float32(0.5) * s * y * y)
    return y


def _emb_body(idxT_hbm, table_hbm, out_hbm,
              idx0, idx1, rows0, rows1, st0, st1,
              g0, g1, o0, o1, i0s, i1s):
    wid = lax.axis_index("s") * _NC + lax.axis_index("c")
    ubase = wid * _UPM * _NM
    iota = lax.iota(jnp.int32, _L)

    def fire_idx(m, idx_v, isem):
        for j in range(_UPM):
            u = ubase + m * _UPM + j
            i1 = u // _NB
            b = u % _NB
            pltpu.async_copy(idxT_hbm.at[i1, pl.ds(b * _BW, _BW)],
                             idx_v.at[j], isem)

    def wait_idx(idx_v, isem):
        for j in range(_UPM):
            pltpu.make_async_copy(
                idxT_hbm.at[0, pl.ds(j * _BW, _BW)],
                idx_v.at[j], isem).wait()

    def fire_gathers(m, idx_v, rows_v, gsem):
        for j in range(_UPM):
            pltpu.async_copy(
                table_hbm.at[idx_v.at[j]],
                rows_v.at[pl.ds(j * _BW, _BW)],
                gsem,
            )

    def wait_gather(rows_v, gsem):
        pltpu.make_async_copy(
            table_hbm.at[pl.ds(0, _UPM * _BW)], rows_v, gsem).wait()

    def compute(rows_v, st_v):
        def group_body(g, c2):
            rowv = g * _L + iota
            j = g // _GROUPS
            rloc = rowv - j * _BW
            diags = []
            colvs = []
            s = None
            for d in range(_D):
                colv = (iota + d) & (_D - 1)
                c = plsc.load_gather(rows_v, [rowv, colv])
                diags.append(c)
                colvs.append(colv)
                s = c * c if s is None else s + c * c
            inv = _rsqrt16(s)
            for d in range(_D):
                plsc.store_scatter(
                    st_v, [jnp.full((_L,), j, jnp.int32), colvs[d], rloc],
                    diags[d] * inv)
            return c2

        lax.fori_loop(0, _UPM * _GROUPS, group_body, 0)

    def fire_out(m, st_v, osem):
        for j in range(_UPM):
            u = ubase + m * _UPM + j
            i1 = u // _NB
            b = u % _NB
            for a in range(_D // 8):
                pltpu.async_copy(
                    st_v.at[j, pl.ds(a * 8, 8)],
                    out_hbm.at[i1, a, b],
                    osem,
                )

    def wait_out(st_v, osem):
        for j in range(_UPM):
            for a in range(_D // 8):
                pltpu.make_async_copy(
                    st_v.at[j, pl.ds(a * 8, 8)],
                    out_hbm.at[0, a, 0],
                    osem,
                ).wait()

    bufs = ((idx0, rows0, st0, g0, o0, i0s), (idx1, rows1, st1, g1, o1, i1s))

    fire_idx(0, idx0, i0s)
    wait_idx(idx0, i0s)
    fire_gathers(0, idx0, rows0, g0)
    fire_idx(1, idx1, i1s)

    def pair_body(i, carry):
        for b in range(2):
            idx_v, rows_v, st_v, gsem, osem, isem = bufs[b]
            nidx_v, nrows_v, _, ngsem, _, nisem = bufs[1 - b]
            m = 2 * i + b

            @pl.when(m + 1 < _NM)
            def _():
                wait_idx(nidx_v, nisem)
                fire_gathers(m + 1, nidx_v, nrows_v, ngsem)

            wait_gather(rows_v, gsem)

            @pl.when(m + 2 < _NM)
            def _():
                fire_idx(m + 2, idx_v, isem)

            @pl.when(m >= 2)
            def _():
                wait_out(st_v, osem)

            compute(rows_v, st_v)
            fire_out(m, st_v, osem)
        return carry

    lax.fori_loop(0, _NM // 2, pair_body, 0)
    wait_out(st0, o0)
    wait_out(st1, o1)


@jax.jit
def _emb(idxT, table):
    mesh = plsc.VectorSubcoreMesh(core_axis_name="c", subcore_axis_name="s")
    f = pl.kernel(
        _emb_body,
        out_type=jax.ShapeDtypeStruct((_B1, _D // 8, _NB, 8, _BW),
                                      jnp.float32),
        mesh=mesh,
        scratch_types=[
            pltpu.VMEM((_UPM, _BW), jnp.int32),
            pltpu.VMEM((_UPM, _BW), jnp.int32),
            pltpu.VMEM((_UPM * _BW, _D), jnp.float32),
            pltpu.VMEM((_UPM * _BW, _D), jnp.float32),
            pltpu.VMEM((_UPM, _D, _BW), jnp.float32),
            pltpu.VMEM((_UPM, _D, _BW), jnp.float32),
            pltpu.SemaphoreType.DMA,
            pltpu.SemaphoreType.DMA,
            pltpu.SemaphoreType.DMA,
            pltpu.SemaphoreType.DMA,
            pltpu.SemaphoreType.DMA,
            pltpu.SemaphoreType.DMA,
        ],
        compiler_params=pltpu.CompilerParams(
            needs_layout_passes=False, use_tc_tiling_on_sc=False),
    )
    return f(idxT, table)


def kernel(inputs, table):
    idxT = inputs.astype(jnp.int32).T
    y5 = _emb(idxT, table)
    return y5.transpose(2, 4, 0, 1, 3).reshape(_B0, _B1, _D)
